# baseline (speedup 1.0000x reference)
import jax
import jax.numpy as jnp
from jax.experimental import pallas as pl
from jax.experimental.pallas import tpu as pltpu

IN_DIM = 512
MID = 1024
H, W = 50, 100
NPIX = H * W
MT = 1000
GRID = 5
GPAD = 104
G_ROWS = 5208
SLICE_ROWS = 1208
NOUT = 64

_OFF = lambda kh, kw: kh * W + kw + 3


def _rpn_kernel(x_ref, w9_hbm, wc_ref, wg_ref, brpn_ref, bc_ref, bg_ref,
                cls_ref, reg_ref, g_ref, w9v_ref, wcrt_ref, sem):
    i = pl.program_id(0)

    def _tap_copy(t):
        return pltpu.make_async_copy(
            w9_hbm.at[pl.ds(t * IN_DIM, IN_DIM), :],
            w9v_ref.at[pl.ds(t * IN_DIM, IN_DIM), :],
            sem.at[t])

    @pl.when(i == 0)
    def _prologue():
        for t in range(9):
            _tap_copy(t).start()

        r = jax.lax.broadcasted_iota(jnp.int32, (IN_DIM, IN_DIM), 0)
        c = jax.lax.broadcasted_iota(jnp.int32, (IN_DIM, IN_DIM), 1)
        eye = (r == c).astype(jnp.bfloat16)
        g_ref[0:GPAD, :] = jnp.zeros((GPAD, IN_DIM), jnp.bfloat16)
        g_ref[GPAD + NPIX:G_ROWS, :] = jnp.zeros((G_ROWS - GPAD - NPIX, IN_DIM),
                                                 jnp.bfloat16)
        for r0 in range(0, H, 2):
            rows = []
            for rr in (r0, r0 + 1):
                xr = x_ref[0, :, rr, :]
                xrt = jax.lax.dot_general(xr, eye, (((0,), (0,)), ((), ())),
                                          preferred_element_type=jnp.float32)
                rows.append(xrt.astype(jnp.bfloat16))
            g_ref[GPAD + r0 * W:GPAD + (r0 + 2) * W, :] = (
                jnp.concatenate(rows, axis=0))

        wcg = jnp.concatenate(
            [wc_ref[...], wg_ref[...],
             jnp.zeros((NOUT - 54, MID), jnp.float32)], axis=0)
        r64 = jax.lax.broadcasted_iota(jnp.int32, (NOUT, NOUT), 0)
        c64 = jax.lax.broadcasted_iota(jnp.int32, (NOUT, NOUT), 1)
        eye64 = (r64 == c64).astype(jnp.bfloat16)
        wcrt = jax.lax.dot_general(wcg.astype(jnp.bfloat16), eye64,
                                   (((0,), (0,)), ((), ())),
                                   preferred_element_type=jnp.float32)
        wcrt_ref[...] = wcrt.astype(jnp.bfloat16)

    base = i * MT
    g = g_ref[pl.ds(base, SLICE_ROWS), :]
    j = jax.lax.broadcasted_iota(jnp.int32, (MT, IN_DIM), 0) % W
    acc = jnp.zeros((MT, MID), dtype=jnp.float32)
    for kh in range(3):
        for kw in range(3):
            lhs = jax.lax.slice_in_dim(g, _OFF(kh, kw), _OFF(kh, kw) + MT,
                                       axis=0)
            if kw == 0:
                lhs = jnp.where(j == 0, jnp.bfloat16(0), lhs)
            elif kw == 2:
                lhs = jnp.where(j == W - 1, jnp.bfloat16(0), lhs)
            t = kh * 3 + kw

            @pl.when(i == 0)
            def _wait_tap(t=t):
                _tap_copy(t).wait()

            rhs = w9v_ref[t * IN_DIM:(t + 1) * IN_DIM, :]
            acc = acc + jnp.dot(lhs, rhs, preferred_element_type=jnp.float32)
    h = (jnp.maximum(acc + brpn_ref[0, :][None, :], 0.0)).astype(jnp.bfloat16)
    out2 = jnp.dot(h, wcrt_ref[...], preferred_element_type=jnp.float32)
    bias = jnp.concatenate(
        [bc_ref[0, :], bg_ref[0, :], jnp.zeros((NOUT - 54,), jnp.float32)],
        axis=0)
    out2 = out2 + bias[None, :]

    col = jax.lax.broadcasted_iota(jnp.int32, (MT, NOUT), 1)
    partner = jnp.where(col % 2 == 0, jnp.roll(out2, -1, axis=1),
                        jnp.roll(out2, 1, axis=1))
    m = jnp.maximum(out2, partner)
    e = jnp.exp(out2 - m)
    soft = e / (e + jnp.exp(partner - m))
    cls_ref[...] = jax.lax.slice_in_dim(soft, 0, 18, axis=1)
    reg_ref[...] = jax.lax.slice_in_dim(out2, 18, 54, axis=1)


def kernel(x, W_rpn, b_rpn, W_cls, b_cls, W_reg, b_reg):
    xb = x.astype(jnp.bfloat16)
    w9 = jnp.transpose(W_rpn, (2, 3, 1, 0)).reshape(9 * IN_DIM, MID)
    w9 = w9.astype(jnp.bfloat16)
    wc = W_cls.reshape(18, MID)
    wg = W_reg.reshape(36, MID)

    cls_out, reg_out = pl.pallas_call(
        _rpn_kernel,
        grid=(GRID,),
        in_specs=[
            pl.BlockSpec((1, IN_DIM, H, W), lambda i: (0, 0, 0, 0)),
            pl.BlockSpec(memory_space=pl.ANY),
            pl.BlockSpec((18, MID), lambda i: (0, 0)),
            pl.BlockSpec((36, MID), lambda i: (0, 0)),
            pl.BlockSpec((1, MID), lambda i: (0, 0)),
            pl.BlockSpec((1, 18), lambda i: (0, 0)),
            pl.BlockSpec((1, 36), lambda i: (0, 0)),
        ],
        out_specs=[pl.BlockSpec((MT, 18), lambda i: (i, 0)),
                   pl.BlockSpec((MT, 36), lambda i: (i, 0))],
        out_shape=[jax.ShapeDtypeStruct((NPIX, 18), jnp.float32),
                   jax.ShapeDtypeStruct((NPIX, 36), jnp.float32)],
        scratch_shapes=[pltpu.VMEM((G_ROWS, IN_DIM), jnp.bfloat16),
                        pltpu.VMEM((9 * IN_DIM, MID), jnp.bfloat16),
                        pltpu.VMEM((MID, NOUT), jnp.bfloat16),
                        pltpu.SemaphoreType.DMA((9,))],
        compiler_params=pltpu.CompilerParams(
            dimension_semantics=("arbitrary",),
        ),
    )(xb, w9, wc, wg, b_rpn[None, :], b_cls[None, :], b_reg[None, :])

    return (cls_out.reshape(NPIX * 9, 2), reg_out.reshape(NPIX * 9, 4))


# speedup vs baseline: 1.0474x; 1.0474x over previous
import jax
import jax.numpy as jnp
from jax.experimental import pallas as pl
from jax.experimental.pallas import tpu as pltpu

IN_DIM = 512
MID = 1024
H, W = 50, 100
NPIX = H * W
MT = 1000
GRID = 5
GPAD = 104
G_ROWS = 5208
SLICE_ROWS = 1208
NOUT = 64

_OFF = lambda kh, kw: kh * W + kw + 3


def _rpn_kernel(g_ref, w9_hbm, wc_ref, wg_ref, brpn_ref, bc_ref, bg_ref,
                cls_ref, reg_ref, w9v_ref, wcrt_ref, sem):
    i = pl.program_id(0)

    def _tap_copy(t):
        return pltpu.make_async_copy(
            w9_hbm.at[pl.ds(t * IN_DIM, IN_DIM), :],
            w9v_ref.at[pl.ds(t * IN_DIM, IN_DIM), :],
            sem.at[t])

    @pl.when(i == 0)
    def _prologue():
        for t in range(9):
            _tap_copy(t).start()
        wcg = jnp.concatenate(
            [wc_ref[...], wg_ref[...],
             jnp.zeros((NOUT - 54, MID), jnp.float32)], axis=0)
        r64 = jax.lax.broadcasted_iota(jnp.int32, (NOUT, NOUT), 0)
        c64 = jax.lax.broadcasted_iota(jnp.int32, (NOUT, NOUT), 1)
        eye64 = (r64 == c64).astype(jnp.bfloat16)
        wcrt = jax.lax.dot_general(wcg.astype(jnp.bfloat16), eye64,
                                   (((0,), (0,)), ((), ())),
                                   preferred_element_type=jnp.float32)
        wcrt_ref[...] = wcrt.astype(jnp.bfloat16)

    base = i * MT
    g = g_ref[pl.ds(base, SLICE_ROWS), :]
    j = jax.lax.broadcasted_iota(jnp.int32, (MT, IN_DIM), 0) % W
    acc = jnp.zeros((MT, MID), dtype=jnp.float32)
    for kh in range(3):
        for kw in range(3):
            lhs = jax.lax.slice_in_dim(g, _OFF(kh, kw), _OFF(kh, kw) + MT,
                                       axis=0)
            if kw == 0:
                lhs = jnp.where(j == 0, jnp.bfloat16(0), lhs)
            elif kw == 2:
                lhs = jnp.where(j == W - 1, jnp.bfloat16(0), lhs)
            t = kh * 3 + kw

            @pl.when(i == 0)
            def _wait_tap(t=t):
                _tap_copy(t).wait()

            rhs = w9v_ref[t * IN_DIM:(t + 1) * IN_DIM, :]
            acc = acc + jnp.dot(lhs, rhs, preferred_element_type=jnp.float32)
    h = (jnp.maximum(acc + brpn_ref[0, :][None, :], 0.0)).astype(jnp.bfloat16)
    out2 = jnp.dot(h, wcrt_ref[...], preferred_element_type=jnp.float32)
    bias = jnp.concatenate(
        [bc_ref[0, :], bg_ref[0, :], jnp.zeros((NOUT - 54,), jnp.float32)],
        axis=0)
    out2 = out2 + bias[None, :]

    col = jax.lax.broadcasted_iota(jnp.int32, (MT, NOUT), 1)
    partner = jnp.where(col % 2 == 0, jnp.roll(out2, -1, axis=1),
                        jnp.roll(out2, 1, axis=1))
    m = jnp.maximum(out2, partner)
    e = jnp.exp(out2 - m)
    soft = e / (e + jnp.exp(partner - m))
    cls_ref[...] = jax.lax.slice_in_dim(soft, 0, 18, axis=1)
    reg_ref[...] = jax.lax.slice_in_dim(out2, 18, 54, axis=1)


def kernel(x, W_rpn, b_rpn, W_cls, b_cls, W_reg, b_reg):
    xt = jnp.transpose(x[0], (1, 2, 0)).reshape(NPIX, IN_DIM)
    g = jnp.pad(xt, ((GPAD, G_ROWS - GPAD - NPIX), (0, 0)))
    g = g.astype(jnp.bfloat16)

    w9 = jnp.transpose(W_rpn, (2, 3, 1, 0)).reshape(9 * IN_DIM, MID)
    w9 = w9.astype(jnp.bfloat16)
    wc = W_cls.reshape(18, MID)
    wg = W_reg.reshape(36, MID)

    cls_out, reg_out = pl.pallas_call(
        _rpn_kernel,
        grid=(GRID,),
        in_specs=[
            pl.BlockSpec((G_ROWS, IN_DIM), lambda i: (0, 0)),
            pl.BlockSpec(memory_space=pl.ANY),
            pl.BlockSpec((18, MID), lambda i: (0, 0)),
            pl.BlockSpec((36, MID), lambda i: (0, 0)),
            pl.BlockSpec((1, MID), lambda i: (0, 0)),
            pl.BlockSpec((1, 18), lambda i: (0, 0)),
            pl.BlockSpec((1, 36), lambda i: (0, 0)),
        ],
        out_specs=[pl.BlockSpec((MT, 18), lambda i: (i, 0)),
                   pl.BlockSpec((MT, 36), lambda i: (i, 0))],
        out_shape=[jax.ShapeDtypeStruct((NPIX, 18), jnp.float32),
                   jax.ShapeDtypeStruct((NPIX, 36), jnp.float32)],
        scratch_shapes=[pltpu.VMEM((9 * IN_DIM, MID), jnp.bfloat16),
                        pltpu.VMEM((MID, NOUT), jnp.bfloat16),
                        pltpu.SemaphoreType.DMA((9,))],
        compiler_params=pltpu.CompilerParams(
            dimension_semantics=("arbitrary",),
        ),
    )(g, w9, wc, wg, b_rpn[None, :], b_cls[None, :], b_reg[None, :])

    return (cls_out.reshape(NPIX * 9, 2), reg_out.reshape(NPIX * 9, 4))


# speedup vs baseline: 1.1578x; 1.1054x over previous
import jax
import jax.numpy as jnp
from jax.experimental import pallas as pl
from jax.experimental.pallas import tpu as pltpu

IN_DIM = 512
MID = 1024
H, W = 50, 100
NPIX = H * W
MT = 1000
GRID = 5
GPAD = 104
G_ROWS = 5208
SLICE_ROWS = 1208
NOUT = 64

_OFF = lambda kh, kw: kh * W + kw + 3


def _rpn_kernel(g_ref, w9_ref, wcr_ref, brpn_ref, bcr_ref, cls_ref, reg_ref):
    i = pl.program_id(0)
    base = i * MT
    g = g_ref[pl.ds(base, SLICE_ROWS), :]
    j = jax.lax.broadcasted_iota(jnp.int32, (MT, IN_DIM), 0) % W
    acc = jnp.zeros((MT, MID), dtype=jnp.float32)
    for kh in range(3):
        for kw in range(3):
            lhs = jax.lax.slice_in_dim(g, _OFF(kh, kw), _OFF(kh, kw) + MT,
                                       axis=0)
            if kw == 0:
                lhs = jnp.where(j == 0, jnp.bfloat16(0), lhs)
            elif kw == 2:
                lhs = jnp.where(j == W - 1, jnp.bfloat16(0), lhs)
            t = kh * 3 + kw
            rhs = w9_ref[t * IN_DIM:(t + 1) * IN_DIM, :]
            acc = acc + jnp.dot(lhs, rhs, preferred_element_type=jnp.float32)
    h = (jnp.maximum(acc + brpn_ref[0, :][None, :], 0.0)).astype(jnp.bfloat16)
    out2 = jnp.dot(h, wcr_ref[...],
                   preferred_element_type=jnp.float32) + bcr_ref[0, :][None, :]

    col = jax.lax.broadcasted_iota(jnp.int32, (MT, NOUT), 1)
    partner = jnp.where(col % 2 == 0, jnp.roll(out2, -1, axis=1),
                        jnp.roll(out2, 1, axis=1))
    m = jnp.maximum(out2, partner)
    e = jnp.exp(out2 - m)
    soft = e / (e + jnp.exp(partner - m))
    cls_ref[...] = jax.lax.slice_in_dim(soft, 0, 18, axis=1)
    reg_ref[...] = jax.lax.slice_in_dim(out2, 18, 54, axis=1)


def kernel(x, W_rpn, b_rpn, W_cls, b_cls, W_reg, b_reg):
    xt = jnp.transpose(x[0], (1, 2, 0)).reshape(NPIX, IN_DIM)
    g = jnp.pad(xt, ((GPAD, G_ROWS - GPAD - NPIX), (0, 0)))
    g = g.astype(jnp.bfloat16)

    w9 = jnp.transpose(W_rpn, (2, 3, 1, 0)).reshape(9 * IN_DIM, MID)
    w9 = w9.astype(jnp.bfloat16)
    wcr = jnp.concatenate([W_cls[:, :, 0, 0], W_reg[:, :, 0, 0]], axis=0)
    wcr = jnp.pad(wcr, ((0, NOUT - 54), (0, 0))).T.astype(jnp.bfloat16)
    bcr = jnp.pad(jnp.concatenate([b_cls, b_reg]), (0, NOUT - 54))

    whole = lambda shape: pl.BlockSpec(shape, lambda i: (0, 0))
    cls_out, reg_out = pl.pallas_call(
        _rpn_kernel,
        grid=(GRID,),
        in_specs=[
            whole((G_ROWS, IN_DIM)),
            whole((9 * IN_DIM, MID)),
            whole((MID, NOUT)),
            whole((1, MID)),
            whole((1, NOUT)),
        ],
        out_specs=[pl.BlockSpec((MT, 18), lambda i: (i, 0)),
                   pl.BlockSpec((MT, 36), lambda i: (i, 0))],
        out_shape=[jax.ShapeDtypeStruct((NPIX, 18), jnp.float32),
                   jax.ShapeDtypeStruct((NPIX, 36), jnp.float32)],
        compiler_params=pltpu.CompilerParams(
            dimension_semantics=("arbitrary",),
        ),
    )(g, w9, wcr, b_rpn[None, :], bcr[None, :])

    return (cls_out.reshape(NPIX * 9, 2), reg_out.reshape(NPIX * 9, 4))


# speedup vs baseline: 1.1758x; 1.0156x over previous
import jax
import jax.numpy as jnp
from jax.experimental import pallas as pl
from jax.experimental.pallas import tpu as pltpu

IN_DIM = 512
MID = 1024
H, W = 50, 100
WP = 104
HP = H + 2
M_TOTAL = H * WP
MT = 1040
GRID = 5
F_ROWS = 4 * MT + 2 * WP + MT + 8
NOUT = 64


def _rpn_kernel(f_ref, w9_ref, wcr_ref, brpn_ref, bcr_ref, out_ref):
    i = pl.program_id(0)
    base = i * MT
    acc = jnp.zeros((MT, MID), dtype=jnp.float32)
    for kh in range(3):
        g = f_ref[pl.ds(base + kh * WP, MT + 8), :]
        for kw in range(3):
            lhs = jax.lax.slice_in_dim(g, kw, kw + MT, axis=0)
            t = kh * 3 + kw
            rhs = w9_ref[t * IN_DIM:(t + 1) * IN_DIM, :]
            acc = acc + jnp.dot(lhs, rhs, preferred_element_type=jnp.float32)
    h = jnp.maximum(acc + brpn_ref[0, :][None, :], 0.0)
    out2 = jnp.dot(h.astype(jnp.bfloat16), wcr_ref[:, :],
                   preferred_element_type=jnp.float32) + bcr_ref[0, :][None, :]
    left = jnp.roll(out2, -1, axis=1)
    right = jnp.roll(out2, 1, axis=1)
    col = jax.lax.broadcasted_iota(jnp.int32, (MT, NOUT), 1)
    partner = jnp.where(col % 2 == 0, left, right)
    m = jnp.maximum(out2, partner)
    e = jnp.exp(out2 - m)
    soft = e / (e + jnp.exp(partner - m))
    out_ref[...] = jnp.where(col < 18, soft, out2)


def kernel(x, W_rpn, b_rpn, W_cls, b_cls, W_reg, b_reg):
    xt = jnp.transpose(x[0], (1, 2, 0))
    xp = jnp.pad(xt, ((1, 1), (1, 3), (0, 0)))
    f = xp.reshape(HP * WP, IN_DIM)
    f = jnp.pad(f, ((0, F_ROWS - HP * WP), (0, 0))).astype(jnp.bfloat16)

    w9 = jnp.transpose(W_rpn, (2, 3, 1, 0)).reshape(9 * IN_DIM, MID)
    w9 = w9.astype(jnp.bfloat16)
    wcr = jnp.concatenate([W_cls[:, :, 0, 0], W_reg[:, :, 0, 0]], axis=0)
    wcr = jnp.pad(wcr, ((0, NOUT - 54), (0, 0))).T.astype(jnp.bfloat16)
    bcr = jnp.pad(jnp.concatenate([b_cls, b_reg]), (0, NOUT - 54))

    whole = lambda shape: pl.BlockSpec(shape, lambda i: (0, 0))
    out = pl.pallas_call(
        _rpn_kernel,
        grid=(GRID,),
        in_specs=[
            whole((F_ROWS, IN_DIM)),
            whole((9 * IN_DIM, MID)),
            whole((MID, NOUT)),
            whole((1, MID)),
            whole((1, NOUT)),
        ],
        out_specs=pl.BlockSpec((MT, NOUT), lambda i: (i, 0)),
        out_shape=jax.ShapeDtypeStruct((M_TOTAL, NOUT), jnp.float32),
        compiler_params=pltpu.CompilerParams(
            dimension_semantics=("arbitrary",),
        ),
    )(f, w9, wcr, b_rpn[None, :], bcr[None, :])

    full = out.reshape(H, WP, NOUT)[:, :W, :]
    cls_out = full[:, :, :18].reshape(H * W * 9, 2)
    reg_out = full[:, :, 18:54].reshape(H * W * 9, 4)
    return (cls_out, reg_out)
